# baseline (speedup 1.0000x reference)
import jax
from jax.experimental import pallas as pl

_E_GRID = 64
_X_GRID = 5


def _copy1_body(a_ref, o_ref):
    o_ref[...] = a_ref[...]


def _copy(a, grid):
    n, d = a.shape
    b = n // grid
    return pl.pallas_call(
        _copy1_body,
        grid=(grid,),
        in_specs=[pl.BlockSpec((b, d), lambda i: (i, 0))],
        out_specs=pl.BlockSpec((b, d), lambda i: (i, 0)),
        out_shape=jax.ShapeDtypeStruct(a.shape, a.dtype),
    )(a)


def kernel(x, edge_index, edge_attr):
    del edge_index
    return (_copy(x, _X_GRID), _copy(edge_attr, _E_GRID))


# speedup vs baseline: 1.0254x; 1.0254x over previous
import jax
from jax.experimental import pallas as pl

_GRID = 25


def _copy_body(x_ref, e_ref, xo_ref, eo_ref):
    xo_ref[...] = x_ref[...]
    eo_ref[...] = e_ref[...]


def kernel(x, edge_index, edge_attr):
    del edge_index
    n_nodes, d_feat = x.shape
    n_edges, d_edge = edge_attr.shape
    bx = n_nodes // _GRID
    be = n_edges // _GRID
    x_out, e_out = pl.pallas_call(
        _copy_body,
        grid=(_GRID,),
        in_specs=[
            pl.BlockSpec((bx, d_feat), lambda i: (i, 0)),
            pl.BlockSpec((be, d_edge), lambda i: (i, 0)),
        ],
        out_specs=[
            pl.BlockSpec((bx, d_feat), lambda i: (i, 0)),
            pl.BlockSpec((be, d_edge), lambda i: (i, 0)),
        ],
        out_shape=[
            jax.ShapeDtypeStruct(x.shape, x.dtype),
            jax.ShapeDtypeStruct(edge_attr.shape, edge_attr.dtype),
        ],
    )(x, edge_attr)
    return (x_out, e_out)
